# baseline (speedup 1.0000x reference)
import functools

import jax
import jax.numpy as jnp
from jax import lax
from jax.experimental import pallas as pl
from jax.experimental.pallas import tpu as pltpu
from jax.experimental.pallas import tpu_sc as plsc

NC = 2
NS = 16
CH = 128


def _round_up(a, m):
    return -(-a // m) * m


def _make_deg_kernel(KD, A, OUT):
    mesh = plsc.VectorSubcoreMesh(core_axis_name="c", subcore_axis_name="s")
    a_sl = A // NS
    o_sl = OUT // NS

    @functools.partial(
        pl.kernel,
        out_type=jax.ShapeDtypeStruct((NC * OUT,), jnp.float32),
        mesh=mesh,
        scratch_types=[
            pltpu.VMEM((KD, CH), jnp.int32),
            pltpu.VMEM((CH,), jnp.float32),
            pltpu.VMEM((o_sl,), jnp.float32),
            pltpu.VMEM((a_sl,), jnp.float32),
            pltpu.VMEM_SHARED((A,), jnp.float32),
        ],
    )
    def deg_kernel(idx_hbm, out_hbm, idx_v, ones_v, stage_v, z_v, acc):
        c = lax.axis_index("c")
        s = lax.axis_index("s")
        wid = c * NS + s
        for i in range(CH // 16):
            ones_v[pl.ds(i * 16, 16)] = jnp.ones((16,), jnp.float32)

        @pl.loop(0, a_sl // 16)
        def _zfill(i):
            z_v[pl.ds(i * 16, 16)] = jnp.zeros((16,), jnp.float32)

        pltpu.sync_copy(idx_hbm.at[pl.ds(wid * KD, KD)], idx_v)
        pltpu.sync_copy(z_v, acc.at[pl.ds(s * a_sl, a_sl)])
        plsc.subcore_barrier()

        @pl.loop(0, KD)
        def _scatter(j):
            pltpu.sync_copy(ones_v, acc.at[idx_v.at[j]], add=True)

        plsc.subcore_barrier()
        pltpu.sync_copy(acc.at[pl.ds(s * o_sl, o_sl)], stage_v)
        pltpu.sync_copy(stage_v, out_hbm.at[pl.ds(c * OUT + s * o_sl, o_sl)])

    return deg_kernel


def _make_agg_kernel(KC, N_pad, H, dtype):
    mesh = plsc.VectorSubcoreMesh(core_axis_name="c", subcore_axis_name="s")
    rows_sl = N_pad // NS
    BI = 32
    NB = KC // BI
    VL = 32 if dtype == jnp.bfloat16 else 16

    @functools.partial(
        pl.kernel,
        out_type=jax.ShapeDtypeStruct((NC * N_pad, H), dtype),
        mesh=mesh,
        scratch_types=[
            pltpu.VMEM((BI, CH), jnp.int32),
            pltpu.VMEM((BI, CH), jnp.int32),
            pltpu.VMEM((2, CH, H), dtype),
            pltpu.SemaphoreType.DMA,
            pltpu.SemaphoreType.DMA,
            pltpu.VMEM((32, H), dtype),
            pltpu.VMEM_SHARED((N_pad, H), dtype),
        ],
    )
    def agg_kernel(hs_hbm, snd_hbm, rcv_hbm, out_hbm,
                   snd_v, rcv_v, rows_v, sem0, sem1, z_v, acc):
        c = lax.axis_index("c")
        s = lax.axis_index("s")
        r0 = s * KC

        for r in range(32):
            for k in range(H // VL):
                z_v[r, pl.ds(k * VL, VL)] = jnp.zeros((VL,), dtype)

        @pl.loop(0, rows_sl // 32)
        def _zinit(i):
            pltpu.sync_copy(z_v, acc.at[pl.ds(s * rows_sl + i * 32, 32)])

        plsc.subcore_barrier()

        @pl.loop(0, NB)
        def _blk(bi):
            pltpu.sync_copy(
                snd_hbm.at[pl.ds(c * NS * KC + r0 + bi * BI, BI)], snd_v)
            pltpu.sync_copy(rcv_hbm.at[pl.ds(r0 + bi * BI, BI)], rcv_v)

            pltpu.async_copy(hs_hbm.at[snd_v.at[0]], rows_v.at[0], sem0)

            @pl.loop(0, BI // 2)
            def _pair(i):
                j0 = 2 * i
                j1 = j0 + 1
                pltpu.async_copy(hs_hbm.at[snd_v.at[j1]], rows_v.at[1], sem1)
                pltpu.make_async_copy(hs_hbm.at[snd_v.at[j0]], rows_v.at[0],
                                      sem0).wait()
                pltpu.sync_copy(rows_v.at[0], acc.at[rcv_v.at[j0]], add=True)

                @pl.when(j1 + 1 < BI)
                def _():
                    pltpu.async_copy(hs_hbm.at[snd_v.at[j1 + 1]],
                                     rows_v.at[0], sem0)

                pltpu.make_async_copy(hs_hbm.at[snd_v.at[j1]], rows_v.at[1],
                                      sem1).wait()
                pltpu.sync_copy(rows_v.at[1], acc.at[rcv_v.at[j1]], add=True)

        plsc.subcore_barrier()

        pltpu.sync_copy(acc.at[pl.ds(s * rows_sl, rows_sl)],
                        out_hbm.at[pl.ds(c * N_pad + s * rows_sl, rows_sl)])

    return agg_kernel


def _mm_body(x_ref, w_ref, b_ref, o_ref):
    o_ref[...] = jnp.dot(x_ref[...], w_ref[...],
                         preferred_element_type=jnp.float32,
                         precision=lax.Precision.HIGHEST) + b_ref[...]


def _scale_body(h_ref, d0_ref, d1_ref, o_ref):
    scale = lax.rsqrt(jnp.maximum(d0_ref[...] + d1_ref[...], 1.0))
    o_ref[...] = h_ref[...] * scale


_SELU_SCALE = 1.0507009873554804934193349852946
_SELU_ALPHA = 1.6732632423543772848170429916717


def _selu(v):
    return _SELU_SCALE * jnp.where(
        v > 0, v, _SELU_ALPHA * (jnp.exp(jnp.minimum(v, 0.0)) - 1.0))


def _post_body(n_real, R, NBLK, GB1, a_ref, d0_ref, d1_ref, wb_ref,
               n1_ref, n2_ref, sm_ref, lg_ref, nscr, cs_scr):
    i = pl.program_id(0)
    scale = lax.rsqrt(jnp.maximum(d0_ref[...] + d1_ref[...], 1.0))
    nodes = _selu(a_ref[...] * scale)
    nscr[pl.ds(i * R, R), :] = nodes

    @pl.when(i < GB1)
    def _():
        n1_ref[...] = nodes

    @pl.when(i >= GB1)
    def _():
        n2_ref[...] = nodes

    @pl.when(i == 0)
    def _():
        cs_scr[...] = jnp.zeros_like(cs_scr)

    @pl.when(i < GB1)
    def _():
        rows = i * R + lax.broadcasted_iota(jnp.int32, (R, 1), 0)
        m = rows < n_real
        cs_scr[...] += jnp.sum(jnp.where(m, nodes, 0.0), axis=0,
                               keepdims=True)

    @pl.when(i == NBLK - 1)
    def _():
        summ = jax.nn.sigmoid(cs_scr[...] / n_real)
        sm_ref[...] = summ

        def body(k, _):
            nb = nscr[pl.ds(k * R, R), :]
            nw = jnp.dot(nb, wb_ref[...],
                         preferred_element_type=jnp.float32,
                         precision=lax.Precision.HIGHEST)
            pk = jnp.where(k < GB1, k * R, n_real + (k - GB1) * R)
            lg_ref[pl.ds(pk, R), :] = jnp.sum(nw * summ, axis=1,
                                              keepdims=True)
            return 0

        lax.fori_loop(0, NBLK, body, 0)


def kernel(x, c_x, edge_index, W, b, Wb):
    N, D = x.shape
    H = W.shape[1]
    E = edge_index.shape[1]
    f32 = jnp.float32

    N_pad = _round_up(N, 2048)
    senders = edge_index[0].astype(jnp.int32)
    receivers = edge_index[1].astype(jnp.int32)

    A = 2 * N_pad + 512
    KD = _round_up(_round_up(2 * E, NC * NS * CH) // (NC * NS * CH), 8)
    LEN = NC * NS * KD * CH
    padd = LEN - 2 * E
    deg_idx = jnp.concatenate([
        senders,
        receivers + N_pad,
        2 * N_pad + (jnp.arange(padd, dtype=jnp.int32) % 512),
    ]).reshape(NC * NS * KD, CH)
    deg_parts = _make_deg_kernel(KD, A, 2 * N_pad)(deg_idx)
    deg_parts = deg_parts.reshape(NC, 2 * N_pad)

    ds0 = deg_parts[0, :N_pad, None]
    ds1 = deg_parts[1, :N_pad, None]
    dr0 = deg_parts[0, N_pad:, None]
    dr1 = deg_parts[1, N_pad:, None]

    xp = jnp.pad(x, ((0, N_pad - N), (0, 0)))
    cxp = jnp.pad(c_x, ((0, N_pad - N), (0, 0)))
    x_cat = jnp.concatenate([xp, cxp], axis=0)
    R = 2048
    nblk = 2 * N_pad // R
    gb1 = N_pad // R
    h0 = pl.pallas_call(
        _mm_body,
        grid=(nblk,),
        in_specs=[
            pl.BlockSpec((R, D), lambda i: (i, 0)),
            pl.BlockSpec((D, H), lambda i: (0, 0)),
            pl.BlockSpec((1, H), lambda i: (0, 0)),
        ],
        out_specs=pl.BlockSpec((R, H), lambda i: (i, 0)),
        out_shape=jax.ShapeDtypeStruct((2 * N_pad, H), f32),
    )(x_cat, W, b[None, :])
    hs = pl.pallas_call(
        _scale_body,
        grid=(nblk,),
        in_specs=[
            pl.BlockSpec((R, H), lambda i: (i, 0)),
            pl.BlockSpec((R, 1), lambda i: (i % gb1, 0)),
            pl.BlockSpec((R, 1), lambda i: (i % gb1, 0)),
        ],
        out_specs=pl.BlockSpec((R, H), lambda i: (i, 0)),
        out_shape=jax.ShapeDtypeStruct((2 * N_pad, H), f32),
    )(h0, ds0, ds1)

    KC = _round_up(_round_up(E, NS * CH) // (NS * CH), 32)
    EP = NS * KC * CH
    padc = EP - E
    pad_rows = N_pad - N
    pad_s = N + (jnp.arange(padc, dtype=jnp.int32) % pad_rows)
    snd_p = jnp.concatenate([senders, pad_s])
    snd2 = jnp.stack([snd_p, snd_p + N_pad]).reshape(NC * NS * KC, CH)
    rcv_p = jnp.concatenate(
        [receivers, N + (jnp.arange(padc, dtype=jnp.int32) % pad_rows)]
    ).reshape(NS * KC, CH)
    agg = _make_agg_kernel(KC, N_pad, H, f32)(hs, snd2, rcv_p)

    nodes1, nodes2, summary2, logits_pk = pl.pallas_call(
        functools.partial(_post_body, N, R, nblk, gb1),
        grid=(nblk,),
        in_specs=[
            pl.BlockSpec((R, H), lambda i: (i, 0)),
            pl.BlockSpec((R, 1), lambda i: (i % gb1, 0)),
            pl.BlockSpec((R, 1), lambda i: (i % gb1, 0)),
            pl.BlockSpec((H, H), lambda i: (0, 0)),
        ],
        out_specs=[
            pl.BlockSpec((R, H), lambda i: (jnp.minimum(i, gb1 - 1), 0)),
            pl.BlockSpec((R, H), lambda i: (jnp.maximum(i - gb1, 0), 0)),
            pl.BlockSpec((1, H), lambda i: (0, 0)),
            pl.BlockSpec((2 * N_pad, 1), lambda i: (0, 0)),
        ],
        out_shape=[
            jax.ShapeDtypeStruct((N, H), f32),
            jax.ShapeDtypeStruct((N, H), f32),
            jax.ShapeDtypeStruct((1, H), f32),
            jax.ShapeDtypeStruct((2 * N_pad, 1), f32),
        ],
        scratch_shapes=[
            pltpu.VMEM((2 * N_pad, H), f32),
            pltpu.VMEM((1, H), f32),
        ],
    )(agg, dr0, dr1, Wb)

    return (nodes1, nodes2, summary2[0]), logits_pk[:2 * N, 0]


# speedup vs baseline: 1.0383x; 1.0383x over previous
import functools

import jax
import jax.numpy as jnp
from jax import lax
from jax.experimental import pallas as pl
from jax.experimental.pallas import tpu as pltpu
from jax.experimental.pallas import tpu_sc as plsc

NC = 2
NS = 16
CH = 128


def _round_up(a, m):
    return -(-a // m) * m


def _make_deg_kernel(KD, A, OUT):
    mesh = plsc.VectorSubcoreMesh(core_axis_name="c", subcore_axis_name="s")
    a_sl = A // NS
    o_sl = OUT // NS

    @functools.partial(
        pl.kernel,
        out_type=jax.ShapeDtypeStruct((NC * OUT,), jnp.float32),
        mesh=mesh,
        scratch_types=[
            pltpu.VMEM((KD, CH), jnp.int32),
            pltpu.VMEM((CH,), jnp.float32),
            pltpu.VMEM((o_sl,), jnp.float32),
            pltpu.VMEM((a_sl,), jnp.float32),
            pltpu.VMEM_SHARED((A,), jnp.float32),
        ],
    )
    def deg_kernel(idx_hbm, out_hbm, idx_v, ones_v, stage_v, z_v, acc):
        c = lax.axis_index("c")
        s = lax.axis_index("s")
        wid = c * NS + s
        for i in range(CH // 16):
            ones_v[pl.ds(i * 16, 16)] = jnp.ones((16,), jnp.float32)

        @pl.loop(0, a_sl // 16)
        def _zfill(i):
            z_v[pl.ds(i * 16, 16)] = jnp.zeros((16,), jnp.float32)

        pltpu.sync_copy(idx_hbm.at[pl.ds(wid * KD, KD)], idx_v)
        pltpu.sync_copy(z_v, acc.at[pl.ds(s * a_sl, a_sl)])
        plsc.subcore_barrier()

        @pl.loop(0, KD)
        def _scatter(j):
            pltpu.sync_copy(ones_v, acc.at[idx_v.at[j]], add=True)

        plsc.subcore_barrier()
        pltpu.sync_copy(acc.at[pl.ds(s * o_sl, o_sl)], stage_v)
        pltpu.sync_copy(stage_v, out_hbm.at[pl.ds(c * OUT + s * o_sl, o_sl)])

    return deg_kernel


def _make_agg_kernel(KC, N_pad, H, dtype):
    mesh = plsc.VectorSubcoreMesh(core_axis_name="c", subcore_axis_name="s")
    rows_sl = N_pad // NS
    BI = 32
    NB = KC // BI
    VL = 32 if dtype == jnp.bfloat16 else 16

    @functools.partial(
        pl.kernel,
        out_type=jax.ShapeDtypeStruct((NC * N_pad, H), dtype),
        mesh=mesh,
        scratch_types=[
            pltpu.VMEM((BI, CH), jnp.int32),
            pltpu.VMEM((BI, CH), jnp.int32),
            pltpu.VMEM((2, CH, H), dtype),
            pltpu.SemaphoreType.DMA,
            pltpu.SemaphoreType.DMA,
            pltpu.VMEM((32, H), dtype),
            pltpu.VMEM_SHARED((N_pad, H), dtype),
        ],
    )
    def agg_kernel(hs_hbm, snd_hbm, rcv_hbm, out_hbm,
                   snd_v, rcv_v, rows_v, sem0, sem1, z_v, acc):
        c = lax.axis_index("c")
        s = lax.axis_index("s")
        r0 = s * KC

        for r in range(32):
            for k in range(H // VL):
                z_v[r, pl.ds(k * VL, VL)] = jnp.zeros((VL,), dtype)

        @pl.loop(0, rows_sl // 32)
        def _zinit(i):
            pltpu.sync_copy(z_v, acc.at[pl.ds(s * rows_sl + i * 32, 32)])

        plsc.subcore_barrier()

        @pl.loop(0, NB)
        def _blk(bi):
            pltpu.sync_copy(
                snd_hbm.at[pl.ds(c * NS * KC + r0 + bi * BI, BI)], snd_v)
            pltpu.sync_copy(rcv_hbm.at[pl.ds(r0 + bi * BI, BI)], rcv_v)

            pltpu.async_copy(hs_hbm.at[snd_v.at[0]], rows_v.at[0], sem0)

            @pl.loop(0, BI // 2)
            def _pair(i):
                j0 = 2 * i
                j1 = j0 + 1
                pltpu.async_copy(hs_hbm.at[snd_v.at[j1]], rows_v.at[1], sem1)
                pltpu.make_async_copy(hs_hbm.at[snd_v.at[j0]], rows_v.at[0],
                                      sem0).wait()
                pltpu.sync_copy(rows_v.at[0], acc.at[rcv_v.at[j0]], add=True)

                @pl.when(j1 + 1 < BI)
                def _():
                    pltpu.async_copy(hs_hbm.at[snd_v.at[j1 + 1]],
                                     rows_v.at[0], sem0)

                pltpu.make_async_copy(hs_hbm.at[snd_v.at[j1]], rows_v.at[1],
                                      sem1).wait()
                pltpu.sync_copy(rows_v.at[1], acc.at[rcv_v.at[j1]], add=True)

        plsc.subcore_barrier()

        pltpu.sync_copy(acc.at[pl.ds(s * rows_sl, rows_sl)],
                        out_hbm.at[pl.ds(c * N_pad + s * rows_sl, rows_sl)])

    return agg_kernel


def _mm_scale_body(x_ref, w_ref, b_ref, d0_ref, d1_ref, o_ref):
    h = jnp.dot(x_ref[...], w_ref[...], preferred_element_type=jnp.float32,
                precision=lax.Precision.HIGHEST) + b_ref[...]
    scale = lax.rsqrt(jnp.maximum(d0_ref[...] + d1_ref[...], 1.0))
    o_ref[...] = h * scale


_SELU_SCALE = 1.0507009873554804934193349852946
_SELU_ALPHA = 1.6732632423543772848170429916717


def _selu(v):
    return _SELU_SCALE * jnp.where(
        v > 0, v, _SELU_ALPHA * (jnp.exp(jnp.minimum(v, 0.0)) - 1.0))


def _post_body(n_real, R, NBLK, GB1, a_ref, d0_ref, d1_ref, wb_ref,
               n1_ref, n2_ref, sm_ref, lg_ref, nscr, cs_scr):
    i = pl.program_id(0)
    scale = lax.rsqrt(jnp.maximum(d0_ref[...] + d1_ref[...], 1.0))
    nodes = _selu(a_ref[...] * scale)
    nscr[pl.ds(i * R, R), :] = nodes

    @pl.when(i < GB1)
    def _():
        n1_ref[...] = nodes

    @pl.when(i >= GB1)
    def _():
        n2_ref[...] = nodes

    @pl.when(i == 0)
    def _():
        cs_scr[...] = jnp.zeros_like(cs_scr)

    @pl.when(i < GB1)
    def _():
        rows = i * R + lax.broadcasted_iota(jnp.int32, (R, 1), 0)
        m = rows < n_real
        cs_scr[...] += jnp.sum(jnp.where(m, nodes, 0.0), axis=0,
                               keepdims=True)

    @pl.when(i == NBLK - 1)
    def _():
        summ = jax.nn.sigmoid(cs_scr[...] / n_real)
        sm_ref[...] = summ

        def body(k, _):
            nb = nscr[pl.ds(k * R, R), :]
            nw = jnp.dot(nb, wb_ref[...],
                         preferred_element_type=jnp.float32,
                         precision=lax.Precision.HIGHEST)
            pk = jnp.where(k < GB1, k * R, n_real + (k - GB1) * R)
            lg_ref[pl.ds(pk, R), :] = jnp.sum(nw * summ, axis=1,
                                              keepdims=True)
            return 0

        lax.fori_loop(0, NBLK, body, 0)


def kernel(x, c_x, edge_index, W, b, Wb):
    N, D = x.shape
    H = W.shape[1]
    E = edge_index.shape[1]
    f32 = jnp.float32

    N_pad = _round_up(N, 2048)
    senders = edge_index[0].astype(jnp.int32)
    receivers = edge_index[1].astype(jnp.int32)

    A = 2 * N_pad + 512
    KD = _round_up(_round_up(2 * E, NC * NS * CH) // (NC * NS * CH), 8)
    LEN = NC * NS * KD * CH
    padd = LEN - 2 * E
    deg_idx = jnp.concatenate([
        senders,
        receivers + N_pad,
        2 * N_pad + (lax.iota(jnp.int32, padd) & 511),
    ]).reshape(NC * NS * KD, CH)
    deg_parts = _make_deg_kernel(KD, A, 2 * N_pad)(deg_idx)
    deg_parts = deg_parts.reshape(NC, 2 * N_pad)

    ds0 = deg_parts[0, :N_pad, None]
    ds1 = deg_parts[1, :N_pad, None]
    dr0 = deg_parts[0, N_pad:, None]
    dr1 = deg_parts[1, N_pad:, None]

    xp = jnp.pad(x, ((0, N_pad - N), (0, 0)))
    cxp = jnp.pad(c_x, ((0, N_pad - N), (0, 0)))
    x_cat = jnp.concatenate([xp, cxp], axis=0)
    R = 2048
    nblk = 2 * N_pad // R
    gb1 = N_pad // R
    hs = pl.pallas_call(
        _mm_scale_body,
        grid=(nblk,),
        in_specs=[
            pl.BlockSpec((R, D), lambda i: (i, 0)),
            pl.BlockSpec((D, H), lambda i: (0, 0)),
            pl.BlockSpec((1, H), lambda i: (0, 0)),
            pl.BlockSpec((R, 1), lambda i: (i % gb1, 0)),
            pl.BlockSpec((R, 1), lambda i: (i % gb1, 0)),
        ],
        out_specs=pl.BlockSpec((R, H), lambda i: (i, 0)),
        out_shape=jax.ShapeDtypeStruct((2 * N_pad, H), f32),
    )(x_cat, W, b[None, :], ds0, ds1)

    KC = _round_up(_round_up(E, NS * CH) // (NS * CH), 32)
    EP = NS * KC * CH
    padc = EP - E
    pad_rows = N_pad - N
    pmask = 1
    while pmask * 2 <= pad_rows:
        pmask *= 2
    pad_s = N + (lax.iota(jnp.int32, padc) & (pmask - 1))
    snd_p = jnp.concatenate([senders, pad_s])
    snd2 = jnp.stack([snd_p, snd_p + N_pad]).reshape(NC * NS * KC, CH)
    rcv_p = jnp.concatenate([receivers, pad_s]).reshape(NS * KC, CH)
    agg = _make_agg_kernel(KC, N_pad, H, f32)(hs, snd2, rcv_p)

    nodes1, nodes2, summary2, logits_pk = pl.pallas_call(
        functools.partial(_post_body, N, R, nblk, gb1),
        grid=(nblk,),
        in_specs=[
            pl.BlockSpec((R, H), lambda i: (i, 0)),
            pl.BlockSpec((R, 1), lambda i: (i % gb1, 0)),
            pl.BlockSpec((R, 1), lambda i: (i % gb1, 0)),
            pl.BlockSpec((H, H), lambda i: (0, 0)),
        ],
        out_specs=[
            pl.BlockSpec((R, H), lambda i: (jnp.minimum(i, gb1 - 1), 0)),
            pl.BlockSpec((R, H), lambda i: (jnp.maximum(i - gb1, 0), 0)),
            pl.BlockSpec((1, H), lambda i: (0, 0)),
            pl.BlockSpec((2 * N_pad, 1), lambda i: (0, 0)),
        ],
        out_shape=[
            jax.ShapeDtypeStruct((N, H), f32),
            jax.ShapeDtypeStruct((N, H), f32),
            jax.ShapeDtypeStruct((1, H), f32),
            jax.ShapeDtypeStruct((2 * N_pad, 1), f32),
        ],
        scratch_shapes=[
            pltpu.VMEM((2 * N_pad, H), f32),
            pltpu.VMEM((1, H), f32),
        ],
    )(agg, dr0, dr1, Wb)

    return (nodes1, nodes2, summary2[0]), logits_pk[:2 * N, 0]
